# Initial kernel scaffold; baseline (speedup 1.0000x reference)
import jax
import jax.numpy as jnp
from jax.experimental import pallas as pl


def kernel(preds, targets):
    raise NotImplementedError("write your pallas kernel here")


# speedup vs baseline: 1.7225x; 1.7225x over previous
import jax
import jax.numpy as jnp
from jax import lax
from jax.experimental import pallas as pl
from jax.experimental.pallas import tpu as pltpu
from jax.experimental.pallas import tpu_sc as plsc

N = 16_777_216
N_BINS = 20
NC = 2
NS = 16
NW = NC * NS
NP = N // NW
CH = 16_384
NCH = NP // CH
VPC = CH // 16


def _hist_body(p_hbm, t_hbm, out_hbm, pbuf, tbuf, hist, sp0, sp1, st0, st1):
    cid = lax.axis_index("c")
    sid = lax.axis_index("s")
    wid = sid * NC + cid
    base = wid * NP

    zero = jnp.zeros((16,), jnp.float32)
    for r in range(2 * N_BINS):
        hist[pl.ds(16 * r, 16)] = zero

    sems_p = (sp0, sp1)
    sems_t = (st0, st1)

    def issue(slot, g):
        off = base + g * CH
        pltpu.async_copy(p_hbm.at[pl.ds(off, CH)], pbuf.at[slot], sems_p[slot])
        pltpu.async_copy(t_hbm.at[pl.ds(off, CH)], tbuf.at[slot], sems_t[slot])

    def wait(slot):
        pltpu.make_async_copy(p_hbm.at[pl.ds(base, CH)], pbuf.at[slot],
                              sems_p[slot]).wait()
        pltpu.make_async_copy(t_hbm.at[pl.ds(base, CH)], tbuf.at[slot],
                              sems_t[slot]).wait()

    lanes = lax.iota(jnp.int32, 16)
    ones = jnp.ones((16,), jnp.float32)

    def consume(slot):
        def vbody(j, c):
            off = j * 16
            p = pbuf[slot, pl.ds(off, 16)]
            t = tbuf[slot, pl.ds(off, 16)]
            b = jnp.minimum((p * jnp.float32(N_BINS)).astype(jnp.int32),
                            N_BINS - 1)
            idx = lax.shift_left(b, 4) + lanes
            plsc.addupdate_scatter(hist, [idx], p - t)
            plsc.addupdate_scatter(hist, [idx + 16 * N_BINS], ones)
            return c

        lax.fori_loop(0, VPC, vbody, 0)

    issue(0, 0)
    issue(1, 1)

    def pair(it, c):
        for s in (0, 1):
            wait(s)
            consume(s)
            issue(s, it * 2 + s + 2)
        return c

    lax.fori_loop(0, NCH // 2 - 1, pair, 0)
    for s in (0, 1):
        wait(s)
        consume(s)

    pltpu.sync_copy(hist, out_hbm.at[wid])


_hist = pl.kernel(
    _hist_body,
    out_type=jax.ShapeDtypeStruct((NW, 2 * N_BINS * 16), jnp.float32),
    mesh=plsc.VectorSubcoreMesh(core_axis_name="c", subcore_axis_name="s"),
    compiler_params=pltpu.CompilerParams(needs_layout_passes=False),
    scratch_types=[
        pltpu.VMEM((2, CH), jnp.float32),
        pltpu.VMEM((2, CH), jnp.float32),
        pltpu.VMEM((2 * N_BINS * 16,), jnp.float32),
        pltpu.SemaphoreType.DMA,
        pltpu.SemaphoreType.DMA,
        pltpu.SemaphoreType.DMA,
        pltpu.SemaphoreType.DMA,
    ],
)


def _finalize_body(x_ref, o_ref):
    x = x_ref[...]
    a = jnp.sum(x, axis=0)
    sd = jnp.sum(a[:N_BINS, :], axis=1, keepdims=True)
    cnt = jnp.sum(a[N_BINS:, :], axis=1, keepdims=True)
    term = jnp.where(cnt > 0, jnp.abs(sd) / jnp.maximum(cnt, 1.0), 0.0)
    o_ref[0, 0] = jnp.sum(term) / jnp.float32(N_BINS)


_finalize = pl.pallas_call(
    _finalize_body,
    out_shape=jax.ShapeDtypeStruct((1, 1), jnp.float32),
    out_specs=pl.BlockSpec(memory_space=pltpu.SMEM),
)


def kernel(preds, targets):
    parts = _hist(preds.reshape(-1), targets.reshape(-1).astype(jnp.float32))
    return _finalize(parts.reshape(NW, 2 * N_BINS, 16))[0, 0]


# speedup vs baseline: 1.7629x; 1.0234x over previous
import jax
import jax.numpy as jnp
from jax import lax
from jax.experimental import pallas as pl
from jax.experimental.pallas import tpu as pltpu
from jax.experimental.pallas import tpu_sc as plsc

N = 16_777_216
N_BINS = 20
NC = 2
NS = 16
NW = NC * NS
NP = N // NW
CH = 16_384
NCH = NP // CH
VPC = CH // 16


UNROLL = 8


def _hist_body(p_hbm, t_hbm, out_hbm, pbuf, tbuf, histd, histc,
               sp0, sp1, st0, st1):
    cid = lax.axis_index("c")
    sid = lax.axis_index("s")
    wid = sid * NC + cid
    base = wid * NP

    zero = jnp.zeros((16,), jnp.float32)
    for r in range(N_BINS):
        histd[pl.ds(16 * r, 16)] = zero
        histc[pl.ds(16 * r, 16)] = zero

    sems_p = (sp0, sp1)
    sems_t = (st0, st1)

    def issue(slot, g):
        off = base + g * CH
        pltpu.async_copy(p_hbm.at[pl.ds(off, CH)], pbuf.at[slot], sems_p[slot])
        pltpu.async_copy(t_hbm.at[pl.ds(off, CH)], tbuf.at[slot], sems_t[slot])

    def wait(slot):
        pltpu.make_async_copy(p_hbm.at[pl.ds(base, CH)], pbuf.at[slot],
                              sems_p[slot]).wait()
        pltpu.make_async_copy(t_hbm.at[pl.ds(base, CH)], tbuf.at[slot],
                              sems_t[slot]).wait()

    lanes = lax.iota(jnp.int32, 16)
    ones = jnp.ones((16,), jnp.float32)

    def consume(slot):
        def vbody(j, c):
            off0 = j * (16 * UNROLL)
            for u in range(UNROLL):
                off = off0 + 16 * u
                p = pbuf[slot, pl.ds(off, 16)]
                t = tbuf[slot, pl.ds(off, 16)]
                b = jnp.minimum((p * jnp.float32(N_BINS)).astype(jnp.int32),
                                N_BINS - 1)
                idx = lax.shift_left(b, 4) + lanes
                plsc.addupdate_scatter(histd, [idx], p - t)
                plsc.addupdate_scatter(histc, [idx], ones)
            return c

        lax.fori_loop(0, VPC // UNROLL, vbody, 0)

    issue(0, 0)
    issue(1, 1)

    def pair(it, c):
        for s in (0, 1):
            wait(s)
            consume(s)
            issue(s, it * 2 + s + 2)
        return c

    lax.fori_loop(0, NCH // 2 - 1, pair, 0)
    for s in (0, 1):
        wait(s)
        consume(s)

    pltpu.sync_copy(histd, out_hbm.at[wid, 0])
    pltpu.sync_copy(histc, out_hbm.at[wid, 1])


_hist = pl.kernel(
    _hist_body,
    out_type=jax.ShapeDtypeStruct((NW, 2, N_BINS * 16), jnp.float32),
    mesh=plsc.VectorSubcoreMesh(core_axis_name="c", subcore_axis_name="s"),
    compiler_params=pltpu.CompilerParams(needs_layout_passes=False),
    scratch_types=[
        pltpu.VMEM((2, CH), jnp.float32),
        pltpu.VMEM((2, CH), jnp.float32),
        pltpu.VMEM((N_BINS * 16,), jnp.float32),
        pltpu.VMEM((N_BINS * 16,), jnp.float32),
        pltpu.SemaphoreType.DMA,
        pltpu.SemaphoreType.DMA,
        pltpu.SemaphoreType.DMA,
        pltpu.SemaphoreType.DMA,
    ],
)


def _finalize_body(x_ref, o_ref):
    x = x_ref[...]
    a = jnp.sum(x, axis=0)
    sd = jnp.sum(a[0], axis=1, keepdims=True)
    cnt = jnp.sum(a[1], axis=1, keepdims=True)
    term = jnp.where(cnt > 0, jnp.abs(sd) / jnp.maximum(cnt, 1.0), 0.0)
    o_ref[0, 0] = jnp.sum(term) / jnp.float32(N_BINS)


_finalize = pl.pallas_call(
    _finalize_body,
    out_shape=jax.ShapeDtypeStruct((1, 1), jnp.float32),
    out_specs=pl.BlockSpec(memory_space=pltpu.SMEM),
)


def kernel(preds, targets):
    parts = _hist(preds.reshape(-1), targets.reshape(-1).astype(jnp.float32))
    return _finalize(parts.reshape(NW, 2, N_BINS, 16))[0, 0]


# speedup vs baseline: 4.9212x; 2.7916x over previous
import jax
import jax.numpy as jnp
from jax import lax
from jax.experimental import pallas as pl
from jax.experimental.pallas import tpu as pltpu
from jax.experimental.pallas import tpu_sc as plsc

N = 16_777_216
N_BINS = 20
NC = 2
NS = 16
NW = NC * NS
NP = N // NW
CH = 16_384
NCH = NP // CH
VPC = CH // 16


UNROLL = 8


def _hist_body(p_hbm, t_hbm, out_hbm, pbuf, tbuf, histd, histc,
               sp0, sp1, st0, st1):
    cid = lax.axis_index("c")
    sid = lax.axis_index("s")
    wid = sid * NC + cid
    base = wid * NP

    zero = jnp.zeros((16,), jnp.float32)
    for r in range(N_BINS):
        histd[pl.ds(16 * r, 16)] = zero
        histc[pl.ds(16 * r, 16)] = zero

    sems_p = (sp0, sp1)
    sems_t = (st0, st1)

    def issue(slot, g):
        off = base + g * CH
        pltpu.async_copy(p_hbm.at[pl.ds(off, CH)], pbuf.at[slot], sems_p[slot])
        pltpu.async_copy(t_hbm.at[pl.ds(off, CH)], tbuf.at[slot], sems_t[slot])

    def wait(slot):
        pltpu.make_async_copy(p_hbm.at[pl.ds(base, CH)], pbuf.at[slot],
                              sems_p[slot]).wait()
        pltpu.make_async_copy(t_hbm.at[pl.ds(base, CH)], tbuf.at[slot],
                              sems_t[slot]).wait()

    lanes = lax.iota(jnp.int32, 16)
    ones = jnp.ones((16,), jnp.float32)

    def consume(slot):
        def vbody(j, c):
            off0 = j * (16 * UNROLL)
            ps, ts = [], []
            for u in range(UNROLL):
                off = off0 + 16 * u
                ps.append(pbuf[slot, pl.ds(off, 16)])
                ts.append(tbuf[slot, pl.ds(off, 16)])
            idxs, diffs = [], []
            for u in range(UNROLL):
                b = jnp.minimum(
                    (ps[u] * jnp.float32(N_BINS)).astype(jnp.int32),
                    N_BINS - 1)
                idxs.append(lax.shift_left(b, 4) + lanes)
                diffs.append(ps[u] - ts[u])
            for u in range(UNROLL):
                plsc.addupdate_scatter(histd, [idxs[u]], diffs[u])
                plsc.addupdate_scatter(histc, [idxs[u]], ones)
            return c

        lax.fori_loop(0, VPC // UNROLL, vbody, 0)

    issue(0, 0)
    issue(1, 1)

    def pair(it, c):
        for s in (0, 1):
            wait(s)
            consume(s)
            issue(s, it * 2 + s + 2)
        return c

    lax.fori_loop(0, NCH // 2 - 1, pair, 0)
    for s in (0, 1):
        wait(s)
        consume(s)

    pltpu.sync_copy(histd, out_hbm.at[wid, 0])
    pltpu.sync_copy(histc, out_hbm.at[wid, 1])


_hist = pl.kernel(
    _hist_body,
    out_type=jax.ShapeDtypeStruct((NW, 2, N_BINS * 16), jnp.float32),
    mesh=plsc.VectorSubcoreMesh(core_axis_name="c", subcore_axis_name="s"),
    compiler_params=pltpu.CompilerParams(needs_layout_passes=False),
    scratch_types=[
        pltpu.VMEM((2, CH), jnp.float32),
        pltpu.VMEM((2, CH), jnp.float32),
        pltpu.VMEM((N_BINS * 16,), jnp.float32),
        pltpu.VMEM((N_BINS * 16,), jnp.float32),
        pltpu.SemaphoreType.DMA,
        pltpu.SemaphoreType.DMA,
        pltpu.SemaphoreType.DMA,
        pltpu.SemaphoreType.DMA,
    ],
)


def _finalize_body(x_ref, o_ref):
    x = x_ref[...]
    a = jnp.sum(x, axis=0)
    sd = jnp.sum(a[0], axis=1, keepdims=True)
    cnt = jnp.sum(a[1], axis=1, keepdims=True)
    term = jnp.where(cnt > 0, jnp.abs(sd) / jnp.maximum(cnt, 1.0), 0.0)
    o_ref[0, 0] = jnp.sum(term) / jnp.float32(N_BINS)


_finalize = pl.pallas_call(
    _finalize_body,
    out_shape=jax.ShapeDtypeStruct((1, 1), jnp.float32),
    out_specs=pl.BlockSpec(memory_space=pltpu.SMEM),
)


def kernel(preds, targets):
    parts = _hist(preds.reshape(-1), targets.reshape(-1).astype(jnp.float32))
    return _finalize(parts.reshape(NW, 2, N_BINS, 16))[0, 0]


# speedup vs baseline: 5.0566x; 1.0275x over previous
import jax
import jax.numpy as jnp
from jax import lax
from jax.experimental import pallas as pl
from jax.experimental.pallas import tpu as pltpu
from jax.experimental.pallas import tpu_sc as plsc

N = 16_777_216
N_BINS = 20
NC = 2
NS = 16
NW = NC * NS
NP = N // NW
CH = 16_384
NCH = NP // CH
VPC = CH // 16


UNROLL = 8


def _hist_body(p_hbm, t_hbm, out_hbm, pbuf, tbuf, histd, histc,
               sp0, sp1, st0, st1):
    cid = lax.axis_index("c")
    sid = lax.axis_index("s")
    wid = sid * NC + cid

    zero = jnp.zeros((16,), jnp.float32)
    for r in range(N_BINS):
        histd[pl.ds(16 * r, 16)] = zero
        histc[pl.ds(16 * r, 16)] = zero

    sems_p = (sp0, sp1)
    sems_t = (st0, st1)

    def issue(slot, g):
        off = (g * NW + wid) * CH
        pltpu.async_copy(p_hbm.at[pl.ds(off, CH)], pbuf.at[slot], sems_p[slot])
        pltpu.async_copy(t_hbm.at[pl.ds(off, CH)], tbuf.at[slot], sems_t[slot])

    def wait(slot):
        pltpu.make_async_copy(p_hbm.at[pl.ds(0, CH)], pbuf.at[slot],
                              sems_p[slot]).wait()
        pltpu.make_async_copy(t_hbm.at[pl.ds(0, CH)], tbuf.at[slot],
                              sems_t[slot]).wait()

    lanes = lax.iota(jnp.int32, 16)
    ones = jnp.ones((16,), jnp.float32)

    def consume(slot):
        def vbody(j, c):
            off0 = j * (16 * UNROLL)
            ps, ts = [], []
            for u in range(UNROLL):
                off = off0 + 16 * u
                ps.append(pbuf[slot, pl.ds(off, 16)])
                ts.append(tbuf[slot, pl.ds(off, 16)])
            idxs, diffs = [], []
            for u in range(UNROLL):
                b = jnp.minimum(ps[u] * jnp.float32(N_BINS),
                                jnp.float32(N_BINS - 1)).astype(jnp.int32)
                idxs.append(lax.shift_left(b, 4) + lanes)
                diffs.append(ps[u] - ts[u])
            for u in range(UNROLL):
                plsc.addupdate_scatter(histd, [idxs[u]], diffs[u])
                plsc.addupdate_scatter(histc, [idxs[u]], ones)
            return c

        lax.fori_loop(0, VPC // UNROLL, vbody, 0)

    issue(0, 0)
    issue(1, 1)

    def pair(it, c):
        for s in (0, 1):
            wait(s)
            consume(s)
            issue(s, it * 2 + s + 2)
        return c

    lax.fori_loop(0, NCH // 2 - 1, pair, 0)
    for s in (0, 1):
        wait(s)
        consume(s)

    pltpu.sync_copy(histd, out_hbm.at[wid, 0])
    pltpu.sync_copy(histc, out_hbm.at[wid, 1])


_hist = pl.kernel(
    _hist_body,
    out_type=jax.ShapeDtypeStruct((NW, 2, N_BINS * 16), jnp.float32),
    mesh=plsc.VectorSubcoreMesh(core_axis_name="c", subcore_axis_name="s"),
    compiler_params=pltpu.CompilerParams(needs_layout_passes=False),
    scratch_types=[
        pltpu.VMEM((2, CH), jnp.float32),
        pltpu.VMEM((2, CH), jnp.float32),
        pltpu.VMEM((N_BINS * 16,), jnp.float32),
        pltpu.VMEM((N_BINS * 16,), jnp.float32),
        pltpu.SemaphoreType.DMA,
        pltpu.SemaphoreType.DMA,
        pltpu.SemaphoreType.DMA,
        pltpu.SemaphoreType.DMA,
    ],
)


def _finalize_body(x_ref, o_ref):
    x = x_ref[...]
    a = jnp.sum(x, axis=0)
    sd = jnp.sum(a[0], axis=1, keepdims=True)
    cnt = jnp.sum(a[1], axis=1, keepdims=True)
    term = jnp.where(cnt > 0, jnp.abs(sd) / jnp.maximum(cnt, 1.0), 0.0)
    o_ref[0, 0] = jnp.sum(term) / jnp.float32(N_BINS)


_finalize = pl.pallas_call(
    _finalize_body,
    out_shape=jax.ShapeDtypeStruct((1, 1), jnp.float32),
    out_specs=pl.BlockSpec(memory_space=pltpu.SMEM),
)


def kernel(preds, targets):
    parts = _hist(preds.reshape(-1), targets.reshape(-1).astype(jnp.float32))
    return _finalize(parts.reshape(NW, 2, N_BINS, 16))[0, 0]
